# baseline (speedup 1.0000x reference)
import jax
import jax.numpy as jnp
from jax.experimental import pallas as pl
from jax.experimental.pallas import tpu as pltpu
from jax.experimental.pallas import tpu_sc as plsc

def _sc_gather(table, idx):
    batch = idx.shape[0]
    n_cols = table.shape[1]
    mesh = plsc.VectorSubcoreMesh(core_axis_name="c", subcore_axis_name="s")
    n_workers = mesh.num_cores * mesh.num_subcores
    n_per = batch // n_workers

    @pl.kernel(
        out_type=jax.ShapeDtypeStruct((batch, n_cols), table.dtype),
        mesh=mesh,
        scratch_types=[
            pltpu.VMEM((n_per,), jnp.int32),
            pltpu.VMEM((n_per, n_cols), table.dtype),
            pltpu.SemaphoreType.DMA,
        ],
    )
    def gather_kernel(x_hbm, i_hbm, o_hbm, idx_v, rows_v, sem):
        wid = jax.lax.axis_index("s") * mesh.num_cores + jax.lax.axis_index("c")
        base = wid * n_per
        pltpu.sync_copy(i_hbm.at[pl.ds(base, n_per)], idx_v)
        pltpu.async_copy(x_hbm.at[idx_v], rows_v, sem).wait()
        pltpu.sync_copy(rows_v, o_hbm.at[pl.ds(base, n_per)])

    return gather_kernel(table, idx)


def _gram_softmax_kernel(a_ref, t_ref, hi_ref, lo_ref, s_acc):
    i = pl.program_id(0)

    @pl.when(i == 0)
    def _():
        s_acc[...] = jnp.zeros_like(s_acc)

    a = a_ref[...].astype(jnp.bfloat16)
    s_acc[...] += jax.lax.dot_general(
        a, a, (((0,), (0,)), ((), ())), preferred_element_type=jnp.float32
    )

    @pl.when(i == pl.num_programs(0) - 1)
    def _():
        s = s_acc[...] * (1.0 / t_ref[0, 0])
        m = jnp.max(s, axis=1, keepdims=True)
        e = jnp.exp(s - m)
        p = e / (jnp.sum(e, axis=1, keepdims=True) + 1e-10)
        hi = p.astype(jnp.bfloat16)
        lo_ref[...] = (p - hi.astype(jnp.float32)).astype(jnp.bfloat16)
        hi_ref[...] = hi


def _scores_kernel(up_ref, hi_ref, lo_ref, out_ref):
    up = up_ref[...].astype(jnp.bfloat16)
    acc = jnp.dot(up, hi_ref[...], preferred_element_type=jnp.float32)
    acc += jnp.dot(up, lo_ref[...], preferred_element_type=jnp.float32)
    out_ref[...] = acc


def kernel(users, adj, temperature):
    n_users, n_items = adj.shape
    batch = users.shape[0]
    block_k = 400

    up = _sc_gather(adj, users.astype(jnp.int32))

    t = jnp.asarray(temperature, jnp.float32).reshape(1, 1)
    hi, lo = pl.pallas_call(
        _gram_softmax_kernel,
        grid=(n_users // block_k,),
        in_specs=[
            pl.BlockSpec((block_k, n_items), lambda i: (i, 0)),
            pl.BlockSpec(memory_space=pltpu.SMEM),
        ],
        out_specs=[
            pl.BlockSpec((n_items, n_items), lambda i: (0, 0)),
            pl.BlockSpec((n_items, n_items), lambda i: (0, 0)),
        ],
        out_shape=[
            jax.ShapeDtypeStruct((n_items, n_items), jnp.bfloat16),
            jax.ShapeDtypeStruct((n_items, n_items), jnp.bfloat16),
        ],
        scratch_shapes=[pltpu.VMEM((n_items, n_items), jnp.float32)],
        compiler_params=pltpu.CompilerParams(vmem_limit_bytes=64 * 1024 * 1024),
    )(adj, t)

    scores = pl.pallas_call(
        _scores_kernel,
        out_shape=jax.ShapeDtypeStruct((batch, n_items), jnp.float32),
    )(up, hi, lo)
    return scores


# speedup vs baseline: 1.0342x; 1.0342x over previous
import jax
import jax.numpy as jnp
from jax.experimental import pallas as pl
from jax.experimental.pallas import tpu as pltpu
from jax.experimental.pallas import tpu_sc as plsc

def _sc_gather(table, idx):
    batch = idx.shape[0]
    n_cols = table.shape[1]
    mesh = plsc.VectorSubcoreMesh(core_axis_name="c", subcore_axis_name="s")
    n_workers = mesh.num_cores * mesh.num_subcores
    n_per = batch // n_workers

    @pl.kernel(
        out_type=jax.ShapeDtypeStruct((batch, n_cols), table.dtype),
        mesh=mesh,
        scratch_types=[
            pltpu.VMEM((n_per,), jnp.int32),
            pltpu.VMEM((n_per, n_cols), table.dtype),
            pltpu.SemaphoreType.DMA,
        ],
    )
    def gather_kernel(x_hbm, i_hbm, o_hbm, idx_v, rows_v, sem):
        wid = jax.lax.axis_index("s") * mesh.num_cores + jax.lax.axis_index("c")
        base = wid * n_per
        pltpu.sync_copy(i_hbm.at[pl.ds(base, n_per)], idx_v)
        pltpu.async_copy(x_hbm.at[idx_v], rows_v, sem).wait()
        pltpu.sync_copy(rows_v, o_hbm.at[pl.ds(base, n_per)])

    return gather_kernel(table, idx)


_SM_CHUNK = 256


def _gram_softmax_kernel(
    a_ref, t_ref, hi_ref, lo_ref, a8, hi_st, lo_st, hi_sems, lo_sems
):
    i = pl.program_id(0)
    nb = a_ref.shape[0]
    a8[pl.ds(i * nb, nb), :] = a_ref[...].astype(jnp.float8_e4m3fn)

    @pl.when(i == pl.num_programs(0) - 1)
    def _():
        n_items = a8.shape[1]
        n_chunks = n_items // _SM_CHUNK
        inv_t = 1.0 / t_ref[0, 0]

        def hbm_copies(c, b):
            rows = pl.ds(c * _SM_CHUNK, _SM_CHUNK)
            return (
                pltpu.make_async_copy(hi_st.at[b], hi_ref.at[rows, :], hi_sems.at[b]),
                pltpu.make_async_copy(lo_st.at[b], lo_ref.at[rows, :], lo_sems.at[b]),
            )

        for c in range(n_chunks):
            b = c % 2
            if c >= 2:
                old_hi, old_lo = hbm_copies(c - 2, b)
                old_hi.wait()
                old_lo.wait()
            cols = pl.ds(c * _SM_CHUNK, _SM_CHUNK)
            s = jax.lax.dot_general(
                a8[:, cols],
                a8[...],
                (((0,), (0,)), ((), ())),
                preferred_element_type=jnp.float32,
            )
            s = s * inv_t
            m = jnp.max(s, axis=1, keepdims=True)
            e = jnp.exp(s - m)
            p = e / (jnp.sum(e, axis=1, keepdims=True) + 1e-10)
            hi = p.astype(jnp.bfloat16)
            hi_st[b] = hi
            lo_st[b] = (p - hi.astype(jnp.float32)).astype(jnp.bfloat16)
            cp_hi, cp_lo = hbm_copies(c, b)
            cp_hi.start()
            cp_lo.start()
        for c in (n_chunks - 2, n_chunks - 1):
            cp_hi, cp_lo = hbm_copies(c, c % 2)
            cp_hi.wait()
            cp_lo.wait()


def _scores_kernel(up_ref, hi_ref, lo_ref, out_ref):
    up = up_ref[...].astype(jnp.bfloat16)
    acc = jnp.dot(up, hi_ref[...], preferred_element_type=jnp.float32)
    acc += jnp.dot(up, lo_ref[...], preferred_element_type=jnp.float32)
    out_ref[...] = acc


def kernel(users, adj, temperature):
    n_users, n_items = adj.shape
    batch = users.shape[0]
    block_k = 400

    up = _sc_gather(adj, users.astype(jnp.int32))

    t = jnp.asarray(temperature, jnp.float32).reshape(1, 1)
    hi, lo = pl.pallas_call(
        _gram_softmax_kernel,
        grid=(n_users // block_k,),
        in_specs=[
            pl.BlockSpec((block_k, n_items), lambda i: (i, 0)),
            pl.BlockSpec(memory_space=pltpu.SMEM),
        ],
        out_specs=[
            pl.BlockSpec(memory_space=pl.ANY),
            pl.BlockSpec(memory_space=pl.ANY),
        ],
        out_shape=[
            jax.ShapeDtypeStruct((n_items, n_items), jnp.bfloat16),
            jax.ShapeDtypeStruct((n_items, n_items), jnp.bfloat16),
        ],
        scratch_shapes=[
            pltpu.VMEM((n_users, n_items), jnp.float8_e4m3fn),
            pltpu.VMEM((2, _SM_CHUNK, n_items), jnp.bfloat16),
            pltpu.VMEM((2, _SM_CHUNK, n_items), jnp.bfloat16),
            pltpu.SemaphoreType.DMA((2,)),
            pltpu.SemaphoreType.DMA((2,)),
        ],
        compiler_params=pltpu.CompilerParams(vmem_limit_bytes=64 * 1024 * 1024),
    )(adj, t)

    scores = pl.pallas_call(
        _scores_kernel,
        out_shape=jax.ShapeDtypeStruct((batch, n_items), jnp.float32),
    )(up, hi, lo)
    return scores


# speedup vs baseline: 1.1308x; 1.0935x over previous
import jax
import jax.numpy as jnp
from jax.experimental import pallas as pl
from jax.experimental.pallas import tpu as pltpu
from jax.experimental.pallas import tpu_sc as plsc

def _sc_gather(table, idx):
    batch = idx.shape[0]
    n_cols = table.shape[1]
    mesh = plsc.VectorSubcoreMesh(core_axis_name="c", subcore_axis_name="s")
    n_workers = mesh.num_cores * mesh.num_subcores
    n_per = batch // n_workers

    @pl.kernel(
        out_type=jax.ShapeDtypeStruct((batch, n_cols), table.dtype),
        mesh=mesh,
        scratch_types=[
            pltpu.VMEM((n_per,), jnp.int32),
            pltpu.VMEM((n_per, n_cols), table.dtype),
            pltpu.SemaphoreType.DMA,
        ],
    )
    def gather_kernel(x_hbm, i_hbm, o_hbm, idx_v, rows_v, sem):
        wid = jax.lax.axis_index("s") * mesh.num_cores + jax.lax.axis_index("c")
        base = wid * n_per
        pltpu.sync_copy(i_hbm.at[pl.ds(base, n_per)], idx_v)
        pltpu.async_copy(x_hbm.at[idx_v], rows_v, sem).wait()
        pltpu.sync_copy(rows_v, o_hbm.at[pl.ds(base, n_per)])

    return gather_kernel(table, idx)


_SM_CHUNK = 256


def _gram_softmax_kernel(
    a_ref, t_ref, hi_ref, lo_ref, a8, hi_st, lo_st, hi_sems, lo_sems
):
    i = pl.program_id(0)
    nb = a_ref.shape[0]
    a8[pl.ds(i * nb, nb), :] = a_ref[...].astype(jnp.float8_e4m3fn)

    @pl.when(i == pl.num_programs(0) - 1)
    def _():
        n_items = a8.shape[1]
        n_chunks = n_items // _SM_CHUNK
        inv_t = 1.0 / t_ref[0, 0]

        def hbm_copies(c, b):
            rows = pl.ds(c * _SM_CHUNK, _SM_CHUNK)
            return (
                pltpu.make_async_copy(hi_st.at[b], hi_ref.at[rows, :], hi_sems.at[b]),
                pltpu.make_async_copy(lo_st.at[b], lo_ref.at[rows, :], lo_sems.at[b]),
            )

        for c in range(n_chunks):
            b = c % 2
            if c >= 2:
                old_hi, old_lo = hbm_copies(c - 2, b)
                old_hi.wait()
                old_lo.wait()
            cols = pl.ds(c * _SM_CHUNK, _SM_CHUNK)
            s = jax.lax.dot_general(
                a8[:, cols],
                a8[...],
                (((0,), (0,)), ((), ())),
                preferred_element_type=jnp.float32,
            )
            s = s * inv_t
            m = jnp.max(s, axis=1, keepdims=True)
            e = jnp.exp(s - m)
            p = e / (jnp.sum(e, axis=1, keepdims=True) + 1e-10)
            hi = p.astype(jnp.bfloat16)
            hi_st[b] = hi
            lo_st[b] = (p - hi.astype(jnp.float32)).astype(jnp.bfloat16)
            cp_hi, cp_lo = hbm_copies(c, b)
            cp_hi.start()
            cp_lo.start()
        for c in (n_chunks - 2, n_chunks - 1):
            cp_hi, cp_lo = hbm_copies(c, c % 2)
            cp_hi.wait()
            cp_lo.wait()


def _scores_kernel(up_ref, hi_ref, lo_ref, out_ref):
    up = up_ref[...].astype(jnp.bfloat16)
    acc = jnp.dot(up, hi_ref[...], preferred_element_type=jnp.float32)
    acc += jnp.dot(up, lo_ref[...], preferred_element_type=jnp.float32)
    out_ref[...] = acc


def kernel(users, adj, temperature):
    n_users, n_items = adj.shape
    batch = users.shape[0]
    block_k = 2000

    up = _sc_gather(adj, users.astype(jnp.int32))

    t = jnp.asarray(temperature, jnp.float32).reshape(1, 1)
    hi, lo = pl.pallas_call(
        _gram_softmax_kernel,
        grid=(n_users // block_k,),
        in_specs=[
            pl.BlockSpec((block_k, n_items), lambda i: (i, 0)),
            pl.BlockSpec(memory_space=pltpu.SMEM),
        ],
        out_specs=[
            pl.BlockSpec(memory_space=pl.ANY),
            pl.BlockSpec(memory_space=pl.ANY),
        ],
        out_shape=[
            jax.ShapeDtypeStruct((n_items, n_items), jnp.bfloat16),
            jax.ShapeDtypeStruct((n_items, n_items), jnp.bfloat16),
        ],
        scratch_shapes=[
            pltpu.VMEM((n_users, n_items), jnp.float8_e4m3fn),
            pltpu.VMEM((2, _SM_CHUNK, n_items), jnp.bfloat16),
            pltpu.VMEM((2, _SM_CHUNK, n_items), jnp.bfloat16),
            pltpu.SemaphoreType.DMA((2,)),
            pltpu.SemaphoreType.DMA((2,)),
        ],
        compiler_params=pltpu.CompilerParams(vmem_limit_bytes=64 * 1024 * 1024),
    )(adj, t)

    scores = pl.pallas_call(
        _scores_kernel,
        out_shape=jax.ShapeDtypeStruct((batch, n_items), jnp.float32),
    )(up, hi, lo)
    return scores


# speedup vs baseline: 1.3123x; 1.1605x over previous
import jax
import jax.numpy as jnp
from jax.experimental import pallas as pl
from jax.experimental.pallas import tpu as pltpu
from jax.experimental.pallas import tpu_sc as plsc

def _sc_gather(table, idx):
    batch = idx.shape[0]
    n_cols = table.shape[1]
    mesh = plsc.VectorSubcoreMesh(core_axis_name="c", subcore_axis_name="s")
    n_workers = mesh.num_cores * mesh.num_subcores
    n_per = batch // n_workers

    @pl.kernel(
        out_type=jax.ShapeDtypeStruct((batch, n_cols), table.dtype),
        mesh=mesh,
        scratch_types=[
            pltpu.VMEM((n_per,), jnp.int32),
            pltpu.VMEM((n_per, n_cols), table.dtype),
            pltpu.SemaphoreType.DMA,
        ],
    )
    def gather_kernel(x_hbm, i_hbm, o_hbm, idx_v, rows_v, sem):
        wid = jax.lax.axis_index("s") * mesh.num_cores + jax.lax.axis_index("c")
        base = wid * n_per
        pltpu.sync_copy(i_hbm.at[pl.ds(base, n_per)], idx_v)
        pltpu.async_copy(x_hbm.at[idx_v], rows_v, sem).wait()
        pltpu.sync_copy(rows_v, o_hbm.at[pl.ds(base, n_per)])

    return gather_kernel(table, idx)


_SM_CHUNK = 256


def _gram_softmax_kernel(a_ref, t_ref, hi_ref, a8, hi_st, hi_sems):
    i = pl.program_id(0)
    nb = a_ref.shape[0]
    a8[pl.ds(i * nb, nb), :] = a_ref[...].astype(jnp.float8_e4m3fn)

    @pl.when(i == pl.num_programs(0) - 1)
    def _():
        n_items = a8.shape[1]
        n_chunks = n_items // _SM_CHUNK
        inv_t = 1.0 / t_ref[0, 0]

        def hbm_copy(c, b):
            rows = pl.ds(c * _SM_CHUNK, _SM_CHUNK)
            return pltpu.make_async_copy(hi_st.at[b], hi_ref.at[rows, :], hi_sems.at[b])

        for c in range(n_chunks):
            b = c % 2
            if c >= 2:
                hbm_copy(c - 2, b).wait()
            cols = pl.ds(c * _SM_CHUNK, _SM_CHUNK)
            s = jax.lax.dot_general(
                a8[:, cols],
                a8[...],
                (((0,), (0,)), ((), ())),
                preferred_element_type=jnp.float32,
            )
            s = s * inv_t
            m = jnp.max(s, axis=1, keepdims=True)
            e = jnp.exp(s - m)
            r = 1.0 / (jnp.sum(e, axis=1, keepdims=True) + 1e-10)
            hi_st[b] = (e * r).astype(jnp.bfloat16)
            hbm_copy(c, b).start()
        for c in (n_chunks - 2, n_chunks - 1):
            hbm_copy(c, c % 2).wait()


def _scores_kernel(up_ref, hi_ref, out_ref, up16):
    @pl.when(pl.program_id(0) == 0)
    def _():
        up16[...] = up_ref[...].astype(jnp.bfloat16)

    out_ref[...] = jnp.dot(
        up16[...], hi_ref[...], preferred_element_type=jnp.float32
    )


def kernel(users, adj, temperature):
    n_users, n_items = adj.shape
    batch = users.shape[0]
    block_k = 2000

    up = _sc_gather(adj, users.astype(jnp.int32))

    t = jnp.asarray(temperature, jnp.float32).reshape(1, 1)
    hi = pl.pallas_call(
        _gram_softmax_kernel,
        grid=(n_users // block_k,),
        in_specs=[
            pl.BlockSpec((block_k, n_items), lambda i: (i, 0)),
            pl.BlockSpec(memory_space=pltpu.SMEM),
        ],
        out_specs=pl.BlockSpec(memory_space=pl.ANY),
        out_shape=jax.ShapeDtypeStruct((n_items, n_items), jnp.bfloat16),
        scratch_shapes=[
            pltpu.VMEM((n_users, n_items), jnp.float8_e4m3fn),
            pltpu.VMEM((2, _SM_CHUNK, n_items), jnp.bfloat16),
            pltpu.SemaphoreType.DMA((2,)),
        ],
        compiler_params=pltpu.CompilerParams(vmem_limit_bytes=64 * 1024 * 1024),
    )(adj, t)

    block_n = 512
    scores = pl.pallas_call(
        _scores_kernel,
        grid=(n_items // block_n,),
        in_specs=[
            pl.BlockSpec((batch, n_items), lambda i: (0, 0)),
            pl.BlockSpec((n_items, block_n), lambda i: (0, i)),
        ],
        out_specs=pl.BlockSpec((batch, block_n), lambda i: (0, i)),
        out_shape=jax.ShapeDtypeStruct((batch, n_items), jnp.float32),
        scratch_shapes=[pltpu.VMEM((batch, n_items), jnp.bfloat16)],
    )(up, hi)
    return scores
